# baseline (speedup 1.0000x reference)
import jax
import jax.numpy as jnp
from jax.experimental import pallas as pl


def _add_body(p_ref, t_ref, o_ref):
    o_ref[...] = p_ref[...] + t_ref[...]


def kernel(patch, pos_table):
    B, P, D = patch.shape
    PD = P * D
    patch2 = patch.reshape(B, PD)
    table2 = pos_table.reshape(1, PD)
    bb = 8
    W = PD // 8
    out = pl.pallas_call(
        _add_body,
        grid=(B // bb, PD // W),
        in_specs=[
            pl.BlockSpec((bb, W), lambda i, j: (i, j)),
            pl.BlockSpec((1, W), lambda i, j: (0, j)),
        ],
        out_specs=pl.BlockSpec((bb, W), lambda i, j: (i, j)),
        out_shape=jax.ShapeDtypeStruct((B, PD), patch.dtype),
    )(patch2, table2)
    return out.reshape(B, P, D)


# speedup vs baseline: 1.3670x; 1.3670x over previous
import jax
import jax.numpy as jnp
from jax.experimental import pallas as pl
from jax.experimental.pallas import tpu as pltpu


def _make_body(B, R, C, NBUF):
    G = B // NBUF

    def body(p_hbm, t_hbm, o_hbm, tv, inb, outb, tsem, insem, outsem):
        pltpu.make_async_copy(t_hbm, tv, tsem).start()
        for k in range(NBUF):
            pltpu.make_async_copy(p_hbm.at[k], inb.at[k], insem.at[k]).start()
        pltpu.make_async_copy(t_hbm, tv, tsem).wait()

        def group(g, _):
            for k in range(NBUF):
                i = g * NBUF + k
                pltpu.make_async_copy(p_hbm.at[i], inb.at[k], insem.at[k]).wait()

                @pl.when(g > 0)
                def _wait_out():
                    pltpu.make_async_copy(outb.at[k], o_hbm.at[0], outsem.at[k]).wait()

                outb[k] = inb[k] + tv[...]
                pltpu.make_async_copy(outb.at[k], o_hbm.at[i], outsem.at[k]).start()

                @pl.when(g < G - 1)
                def _prefetch():
                    pltpu.make_async_copy(
                        p_hbm.at[(g + 1) * NBUF + k], inb.at[k], insem.at[k]
                    ).start()

            return 0

        jax.lax.fori_loop(0, G, group, 0)
        for k in range(NBUF):
            pltpu.make_async_copy(outb.at[k], o_hbm.at[0], outsem.at[k]).wait()

    return body


def kernel(patch, pos_table):
    B, P, D = patch.shape
    PD = P * D
    R = 8
    C = PD // R
    NBUF = 8
    patch3 = patch.reshape(B, R, C)
    table3 = pos_table.reshape(R, C)
    out = pl.pallas_call(
        _make_body(B, R, C, NBUF),
        in_specs=[
            pl.BlockSpec(memory_space=pltpu.HBM),
            pl.BlockSpec(memory_space=pltpu.HBM),
        ],
        out_specs=pl.BlockSpec(memory_space=pltpu.HBM),
        out_shape=jax.ShapeDtypeStruct((B, R, C), patch.dtype),
        scratch_shapes=[
            pltpu.VMEM((R, C), jnp.float32),
            pltpu.VMEM((NBUF, R, C), jnp.float32),
            pltpu.VMEM((NBUF, R, C), jnp.float32),
            pltpu.SemaphoreType.DMA,
            pltpu.SemaphoreType.DMA((NBUF,)),
            pltpu.SemaphoreType.DMA((NBUF,)),
        ],
    )(patch3, table3)
    return out.reshape(B, P, D)


# speedup vs baseline: 1.3830x; 1.0117x over previous
import jax
import jax.numpy as jnp
from jax.experimental import pallas as pl
from jax.experimental.pallas import tpu as pltpu


def _make_body(B, R, C, CH, NBUF):
    steps = B // CH
    G = steps // NBUF

    def body(p_hbm, t_hbm, o_hbm, tv, inb, outb, tsem, insem, outsem):
        pltpu.make_async_copy(t_hbm, tv, tsem).start()
        for k in range(NBUF):
            pltpu.make_async_copy(
                p_hbm.at[pl.ds(k * CH, CH)], inb.at[k], insem.at[k]
            ).start()
        pltpu.make_async_copy(t_hbm, tv, tsem).wait()

        def group(g, _):
            for k in range(NBUF):
                i = g * NBUF + k
                pltpu.make_async_copy(
                    p_hbm.at[pl.ds(i * CH, CH)], inb.at[k], insem.at[k]
                ).wait()

                @pl.when(g > 0)
                def _wait_out():
                    pltpu.make_async_copy(
                        outb.at[k], o_hbm.at[pl.ds(0, CH)], outsem.at[k]
                    ).wait()

                outb[k] = inb[k] + tv[None]
                pltpu.make_async_copy(
                    outb.at[k], o_hbm.at[pl.ds(i * CH, CH)], outsem.at[k]
                ).start()

                @pl.when(g < G - 1)
                def _prefetch():
                    ni = (g + 1) * NBUF + k
                    pltpu.make_async_copy(
                        p_hbm.at[pl.ds(ni * CH, CH)], inb.at[k], insem.at[k]
                    ).start()

            return 0

        jax.lax.fori_loop(0, G, group, 0)
        for k in range(NBUF):
            pltpu.make_async_copy(
                outb.at[k], o_hbm.at[pl.ds(0, CH)], outsem.at[k]
            ).wait()

    return body


def kernel(patch, pos_table):
    B, P, D = patch.shape
    PD = P * D
    R = 8
    C = PD // R
    CH = 2
    NBUF = 8
    patch3 = patch.reshape(B, R, C)
    table3 = pos_table.reshape(R, C)
    out = pl.pallas_call(
        _make_body(B, R, C, CH, NBUF),
        in_specs=[
            pl.BlockSpec(memory_space=pltpu.HBM),
            pl.BlockSpec(memory_space=pltpu.HBM),
        ],
        out_specs=pl.BlockSpec(memory_space=pltpu.HBM),
        out_shape=jax.ShapeDtypeStruct((B, R, C), patch.dtype),
        scratch_shapes=[
            pltpu.VMEM((R, C), jnp.float32),
            pltpu.VMEM((NBUF, CH, R, C), jnp.float32),
            pltpu.VMEM((NBUF, CH, R, C), jnp.float32),
            pltpu.SemaphoreType.DMA,
            pltpu.SemaphoreType.DMA((NBUF,)),
            pltpu.SemaphoreType.DMA((NBUF,)),
        ],
    )(patch3, table3)
    return out.reshape(B, P, D)
